# Initial kernel scaffold; baseline (speedup 1.0000x reference)
import jax
import jax.numpy as jnp
from jax.experimental import pallas as pl


def kernel(x, edge_index, W1, a_src1, a_dst1, b1, W2, a_src2, a_dst2, b2):
    raise NotImplementedError("write your pallas kernel here")


# speedup vs baseline: 10.2482x; 10.2482x over previous
import functools

import jax
import jax.numpy as jnp
from jax import lax
from jax.experimental import pallas as pl
from jax.experimental.pallas import tpu as pltpu
from jax.experimental.pallas import tpu_sc as plsc

N = 50000
E = 800000
F_IN = 1433
H1 = 8
C1 = 16
C_OUT = 7

NWORK = 32
P1 = 4
NPR1 = 400
NPR2 = 1600
NPAD = 51200
CH1 = 3200
CH2 = 6400
G1 = 32
G2 = 64


def _lane_bcast(x, i):
    idx = jnp.full((16, 1), i, jnp.int32)
    dnums = lax.GatherDimensionNumbers(offset_dims=(), collapsed_slice_dims=(0,),
                                       start_index_map=(0,))
    return lax.gather(x, idx, dnums, (1,),
                      mode=lax.GatherScatterMode.PROMISE_IN_BOUNDS)


def _mm_kernel(x_ref, w_ref, o_ref, *, dpad):
    r = jnp.dot(x_ref[...], w_ref[...], preferred_element_type=jnp.float32)
    if dpad:
        r = jnp.concatenate(
            [r, jnp.zeros((r.shape[0], dpad), jnp.float32)], axis=1)
    o_ref[...] = r


def _tc_matmul(x, w, bm, dpad=0):
    m, k = x.shape
    _, n = w.shape
    grid = (m + bm - 1) // bm
    return pl.pallas_call(
        functools.partial(_mm_kernel, dpad=dpad),
        grid=(grid,),
        in_specs=[pl.BlockSpec((bm, k), lambda i: (i, 0)),
                  pl.BlockSpec((k, n), lambda i: (0, 0))],
        out_specs=pl.BlockSpec((bm, n + dpad), lambda i: (i, 0)),
        out_shape=jax.ShapeDtypeStruct((m, n + dpad), jnp.float32),
    )(x, w)


def _sc_layer1(src_h, dst_h, hcat_h, ad8_h, b1_h, h2_h,
               accm, accp, adl, b1v, dbuf, sbuf, seld, sels, rows, sem):
    wid = lax.axis_index("s") * 2 + lax.axis_index("c")
    iota = lax.iota(jnp.int32, 16)
    zf = jnp.zeros((16,), jnp.float32)
    zi = jnp.zeros((16,), jnp.int32)

    pltpu.sync_copy(b1_h, b1v)

    def zero_sel(i, _):
        seld[pl.ds(i * 16, 16)] = zi
        sels[pl.ds(i * 16, 16)] = zi
        return 0
    lax.fori_loop(0, CH1 // 16, zero_sel, 0)

    def pass_body(p, _):
        base = (wid * P1 + p) * NPR1
        pltpu.sync_copy(ad8_h.at[pl.ds(base, NPR1), :], adl)

        def zero_m(i, _):
            accm[pl.ds(i * 16, 16)] = zf
            return 0
        lax.fori_loop(0, NPR1 * 128 // 16, zero_m, 0)

        def zero_p(i, _):
            accp[pl.ds(i * 16, 16)] = zf
            return 0
        lax.fori_loop(0, NPR1, zero_p, 0)

        def chunk_body(c, _):
            pltpu.sync_copy(src_h.at[pl.ds(c * CH1, CH1)], sbuf)
            pltpu.sync_copy(dst_h.at[pl.ds(c * CH1, CH1)], dbuf)

            def scan_body(v, nselv):
                dv = dbuf[pl.ds(v * 16, 16)]
                within = (dv >= base) & (dv < base + NPR1)
                sv = sbuf[pl.ds(v * 16, 16)]
                incl = plsc.cumsum(within.astype(jnp.int32))
                pos = nselv + incl - 1
                plsc.store_scatter(seld, [pos], dv - base, mask=within)
                plsc.store_scatter(sels, [pos], sv, mask=within)
                return nselv + _lane_bcast(incl, 15)
            nselv = lax.fori_loop(0, CH1 // 16, scan_body, zi)
            nsel = jnp.max(nselv)

            def batch_body(b, _):
                gb = b * G1
                pltpu.async_copy(hcat_h.at[sels.at[pl.ds(gb, G1)]], rows,
                                 sem).wait()

                def vec_body(v, _):
                    off = gb + v * 16
                    dloc = jnp.clip(seld[pl.ds(off, 16)], 0, NPR1 - 1)
                    valid = (off + iota) < nsel
                    lid = iota + v * 16
                    for j in range(H1):
                        asj = plsc.load_gather(
                            rows, [lid, jnp.full((16,), 128 + j, jnp.int32)])
                        adj = plsc.load_gather(
                            adl, [dloc, jnp.full((16,), j, jnp.int32)])
                        e = asj + adj
                        e = jnp.where(e > 0, e, 0.2 * e)
                        pj = jnp.exp(e)
                        plsc.addupdate_scatter(accp, [dloc * 16 + j], pj,
                                               mask=valid)
                        for cc in range(C1):
                            col = j * 16 + cc
                            hv = plsc.load_gather(
                                rows, [lid, jnp.full((16,), col, jnp.int32)])
                            plsc.addupdate_scatter(accm, [dloc * 128 + col],
                                                   pj * hv, mask=valid)
                    return 0
                lax.fori_loop(0, G1 // 16, vec_body, 0)
                return 0
            lax.fori_loop(0, (nsel + G1 - 1) // G1, batch_body, 0)
            return 0
        lax.fori_loop(0, E // CH1, chunk_body, 0)

        def fin_body(n, _):
            for j in range(H1):
                dn = plsc.load_gather(accp, [jnp.full((16,), 1, jnp.int32)
                                             * (n * 16 + j)])
                rv = 1.0 / (dn + 1e-16)
                bv = b1v[pl.ds(j * 16, 16)]
                val = accm[pl.ds(n * 128 + j * 16, 16)] * rv + bv
                res = jnp.where(val > 0, val, jnp.exp(val) - 1.0)
                accm[pl.ds(n * 128 + j * 16, 16)] = res
            return 0
        lax.fori_loop(0, NPR1, fin_body, 0)

        pltpu.sync_copy(accm, h2_h.at[pl.ds(base * 128, NPR1 * 128)])
        return 0
    lax.fori_loop(0, P1, pass_body, 0)


def _sc_layer2(src_h, dst_h, gcat_h, ad2_h, b2_h, out_h,
               acc, adl, b2v, dbuf, sbuf, seld, sels, rows, sem):
    wid = lax.axis_index("s") * 2 + lax.axis_index("c")
    iota = lax.iota(jnp.int32, 16)
    zf = jnp.zeros((16,), jnp.float32)
    zi = jnp.zeros((16,), jnp.int32)
    base = wid * NPR2

    pltpu.sync_copy(b2_h, b2v)
    pltpu.sync_copy(ad2_h.at[pl.ds(base, NPR2)], adl)

    def zero_sel(i, _):
        seld[pl.ds(i * 16, 16)] = zi
        sels[pl.ds(i * 16, 16)] = zi
        return 0
    lax.fori_loop(0, CH2 // 16, zero_sel, 0)

    def zero_a(i, _):
        acc[pl.ds(i * 16, 16)] = zf
        return 0
    lax.fori_loop(0, NPR2, zero_a, 0)

    def chunk_body(c, _):
        pltpu.sync_copy(src_h.at[pl.ds(c * CH2, CH2)], sbuf)
        pltpu.sync_copy(dst_h.at[pl.ds(c * CH2, CH2)], dbuf)

        def scan_body(v, nselv):
            dv = dbuf[pl.ds(v * 16, 16)]
            within = (dv >= base) & (dv < base + NPR2)
            sv = sbuf[pl.ds(v * 16, 16)]
            incl = plsc.cumsum(within.astype(jnp.int32))
            pos = nselv + incl - 1
            plsc.store_scatter(seld, [pos], dv - base, mask=within)
            plsc.store_scatter(sels, [pos], sv, mask=within)
            return nselv + _lane_bcast(incl, 15)
        nselv = lax.fori_loop(0, CH2 // 16, scan_body, zi)
        nsel = jnp.max(nselv)

        def batch_body(b, _):
            gb = b * G2
            pltpu.async_copy(gcat_h.at[sels.at[pl.ds(gb, G2)]], rows,
                             sem).wait()

            def vec_body(v, _):
                off = gb + v * 16
                dloc = jnp.clip(seld[pl.ds(off, 16)], 0, NPR2 - 1)
                valid = (off + iota) < nsel
                lid = iota + v * 16
                asv = plsc.load_gather(
                    rows, [lid, jnp.full((16,), 7, jnp.int32)])
                adv = plsc.load_gather(adl, [dloc])
                e = asv + adv
                e = jnp.where(e > 0, e, 0.2 * e)
                p = jnp.exp(e)
                plsc.addupdate_scatter(acc, [dloc * 16 + 7], p, mask=valid)
                for cc in range(C_OUT):
                    gv = plsc.load_gather(
                        rows, [lid, jnp.full((16,), cc, jnp.int32)])
                    plsc.addupdate_scatter(acc, [dloc * 16 + cc], p * gv,
                                           mask=valid)
                return 0
            lax.fori_loop(0, G2 // 16, vec_body, 0)
            return 0
        lax.fori_loop(0, (nsel + G2 - 1) // G2, batch_body, 0)
        return 0
    lax.fori_loop(0, E // CH2, chunk_body, 0)

    bvec = b2v[pl.ds(0, 16)]

    def fin_body(n, _):
        dn = plsc.load_gather(acc, [jnp.full((16,), 1, jnp.int32)
                                    * (n * 16 + 7)])
        rv = 1.0 / (dn + 1e-16)
        acc[pl.ds(n * 16, 16)] = acc[pl.ds(n * 16, 16)] * rv + bvec
        return 0
    lax.fori_loop(0, NPR2, fin_body, 0)

    pltpu.sync_copy(acc, out_h.at[pl.ds(base * 16, NPR2 * 16)])


@functools.lru_cache(maxsize=None)
def _sc_calls():
    mesh = plsc.VectorSubcoreMesh(core_axis_name="c", subcore_axis_name="s")
    sc1 = pl.kernel(
        _sc_layer1,
        out_type=jax.ShapeDtypeStruct((NPAD * 128,), jnp.float32),
        mesh=mesh,
        scratch_types=[
            pltpu.VMEM((NPR1 * 128,), jnp.float32),
            pltpu.VMEM((NPR1 * 16,), jnp.float32),
            pltpu.VMEM((NPR1, 8), jnp.float32),
            pltpu.VMEM((128,), jnp.float32),
            pltpu.VMEM((CH1,), jnp.int32),
            pltpu.VMEM((CH1,), jnp.int32),
            pltpu.VMEM((CH1,), jnp.int32),
            pltpu.VMEM((CH1,), jnp.int32),
            pltpu.VMEM((G1, 256), jnp.float32),
            pltpu.SemaphoreType.DMA,
        ],
        compiler_params=pltpu.CompilerParams(needs_layout_passes=False),
    )
    sc2 = pl.kernel(
        _sc_layer2,
        out_type=jax.ShapeDtypeStruct((NPAD * 16,), jnp.float32),
        mesh=mesh,
        scratch_types=[
            pltpu.VMEM((NPR2 * 16,), jnp.float32),
            pltpu.VMEM((NPR2,), jnp.float32),
            pltpu.VMEM((16,), jnp.float32),
            pltpu.VMEM((CH2,), jnp.int32),
            pltpu.VMEM((CH2,), jnp.int32),
            pltpu.VMEM((CH2,), jnp.int32),
            pltpu.VMEM((CH2,), jnp.int32),
            pltpu.VMEM((G2, 128), jnp.float32),
            pltpu.SemaphoreType.DMA,
        ],
        compiler_params=pltpu.CompilerParams(needs_layout_passes=False),
    )
    return sc1, sc2


def kernel(x, edge_index, W1, a_src1, a_dst1, b1, W2, a_src2, a_dst2, b2):
    src = edge_index[0]
    dst = edge_index[1]

    eye = jnp.eye(H1, dtype=jnp.float32)
    p_src = (eye[:, None, :] * a_src1[:, :, None]).reshape(H1 * C1, H1)
    p_dst = (eye[:, None, :] * a_dst1[:, :, None]).reshape(H1 * C1, H1)
    wcat1 = jnp.concatenate([W1, W1 @ p_src, W1 @ p_dst], axis=1)

    sc1_call, sc2_call = _sc_calls()

    hcat = _tc_matmul(x, wcat1, 512, dpad=112)
    ad8 = jnp.pad(hcat[:, 136:144], ((0, NPAD - N), (0, 0)))
    h2_flat = sc1_call(src, dst, hcat, ad8, b1)
    h2 = h2_flat.reshape(NPAD, 128)

    wcat2 = jnp.concatenate(
        [W2, W2 @ a_src2[0][:, None], W2 @ a_dst2[0][:, None],
         jnp.zeros((H1 * C1, 119), jnp.float32)], axis=1)
    gcat = _tc_matmul(h2, wcat2, 512)
    ad2 = gcat[:, 8]
    b2p = jnp.zeros((16,), jnp.float32).at[:C_OUT].set(b2)

    out_flat = sc2_call(src, dst, gcat, ad2, b2p)
    return out_flat.reshape(NPAD, 16)[:N, :C_OUT]


# speedup vs baseline: 20.0661x; 1.9580x over previous
import functools

import jax
import jax.numpy as jnp
from jax import lax
from jax.experimental import pallas as pl
from jax.experimental.pallas import tpu as pltpu
from jax.experimental.pallas import tpu_sc as plsc

N = 50000
E = 800000
F_IN = 1433
H1 = 8
C1 = 16
C_OUT = 7

NWORK = 32
NPT = 1600
P1 = 5
NPR1 = 320
NPAD = 51200
CHA = 16000
CAPA = 32768
FB = 4096
EPAD = 806912
CL = 6400
CAPB = 8192
G = 64


def _lane_bcast(x, i):
    idx = jnp.full((16, 1), i, jnp.int32)
    dnums = lax.GatherDimensionNumbers(offset_dims=(), collapsed_slice_dims=(0,),
                                       start_index_map=(0,))
    return lax.gather(x, idx, dnums, (1,),
                      mode=lax.GatherScatterMode.PROMISE_IN_BOUNDS)


def _mm_kernel(x_ref, w_ref, o_ref, *, dpad):
    r = jnp.dot(x_ref[...], w_ref[...], preferred_element_type=jnp.float32)
    if dpad:
        r = jnp.concatenate(
            [r, jnp.zeros((r.shape[0], dpad), jnp.float32)], axis=1)
    o_ref[...] = r


def _tc_matmul(x, w, bm, dpad=0):
    m, k = x.shape
    _, n = w.shape
    grid = (m + bm - 1) // bm
    return pl.pallas_call(
        functools.partial(_mm_kernel, dpad=dpad),
        grid=(grid,),
        in_specs=[pl.BlockSpec((bm, k), lambda i: (i, 0)),
                  pl.BlockSpec((k, n), lambda i: (0, 0))],
        out_specs=pl.BlockSpec((bm, n + dpad), lambda i: (i, 0)),
        out_shape=jax.ShapeDtypeStruct((m, n + dpad), jnp.float32),
    )(x, w)


def _sc_partition(src_h, dst_h, sel_h, cnt_h, dbuf, sbuf, ring, cntv, sem):
    wid = lax.axis_index("s") * 2 + lax.axis_index("c")
    base = wid * NPT
    zi = jnp.zeros((16,), jnp.int32)

    def chunk_body(c, carry):
        nselv, flushed = carry
        pltpu.sync_copy(src_h.at[pl.ds(c * CHA, CHA)], sbuf)
        pltpu.sync_copy(dst_h.at[pl.ds(c * CHA, CHA)], dbuf)

        def scan_body(v, nselv):
            dv = dbuf[pl.ds(v * 16, 16)]
            within = (dv >= base) & (dv < base + NPT)
            sv = sbuf[pl.ds(v * 16, 16)]
            incl = plsc.cumsum(within.astype(jnp.int32))
            pos = (nselv + incl - 1) & (CAPA - 1)
            plsc.store_scatter(ring, [pos], sv * 2048 + (dv - base),
                               mask=within)
            return nselv + _lane_bcast(incl, 15)
        nselv = lax.fori_loop(0, CHA // 16, scan_body, nselv)
        nsel = jnp.max(nselv)

        def flush_body(f, flushed):
            fo = pl.multiple_of(flushed, FB)
            pltpu.sync_copy(ring.at[pl.ds(pl.multiple_of(fo & (CAPA - 1), FB), FB)],
                            sel_h.at[pl.ds(wid * EPAD + fo, FB)])
            return flushed + FB
        flushed = lax.fori_loop(0, (nsel - flushed) // FB, flush_body, flushed)
        return nselv, flushed
    nselv, flushed = lax.fori_loop(0, E // CHA, chunk_body, (zi, 0))
    nsel = jnp.max(nselv)

    def flush_tail(f, flushed):
        fo = pl.multiple_of(flushed, FB)
        pltpu.sync_copy(ring.at[pl.ds(pl.multiple_of(fo & (CAPA - 1), FB), FB)],
                        sel_h.at[pl.ds(wid * EPAD + fo, FB)])
        return flushed + FB
    lax.fori_loop(0, (nsel - flushed + FB - 1) // FB, flush_tail, flushed)

    cntv[pl.ds(0, 16)] = nselv
    pltpu.sync_copy(cntv, cnt_h.at[pl.ds(wid * 16, 16)])


def _sc_layer1(sel_h, cnt_h, hcat_h, ad8_h, b1_h, h2_h,
               accm, accp, adl, b1v, listb, sels, seld, rows, cntv, sem):
    wid = lax.axis_index("s") * 2 + lax.axis_index("c")
    iota = lax.iota(jnp.int32, 16)
    zf = jnp.zeros((16,), jnp.float32)
    zi = jnp.zeros((16,), jnp.int32)

    pltpu.sync_copy(b1_h, b1v)
    pltpu.sync_copy(cnt_h.at[pl.ds(wid * 16, 16)], cntv)
    cntvec = cntv[pl.ds(0, 16)]
    cnt = jnp.max(cntvec)

    def zero_sel(i, _):
        sels[pl.ds(i * 16, 16)] = zi
        seld[pl.ds(i * 16, 16)] = zi
        return 0
    lax.fori_loop(0, CAPB // 16, zero_sel, 0)

    def process_batch(proc, nsel):
        gb = pl.multiple_of(pl.multiple_of(proc, G) & (CAPB - 1), G)
        pltpu.async_copy(hcat_h.at[sels.at[pl.ds(gb, G)]], rows, sem).wait()

        def vec_body(v, _):
            off = gb + v * 16
            dloc = jnp.clip(seld[pl.ds(off, 16)], 0, NPR1 - 1)
            valid = (proc + v * 16 + iota) < nsel
            lid = iota + v * 16
            for j in range(H1):
                asj = plsc.load_gather(
                    rows, [lid, jnp.full((16,), 128 + j, jnp.int32)])
                adj = plsc.load_gather(
                    adl, [dloc, jnp.full((16,), j, jnp.int32)])
                e = asj + adj
                e = jnp.where(e > 0, e, 0.2 * e)
                pj = jnp.exp(e)
                plsc.addupdate_scatter(accp, [dloc * 16 + j], pj, mask=valid)
                for cc in range(C1):
                    col = j * 16 + cc
                    hv = plsc.load_gather(
                        rows, [lid, jnp.full((16,), col, jnp.int32)])
                    plsc.addupdate_scatter(accm, [dloc * 128 + col],
                                           pj * hv, mask=valid)
            return 0
        lax.fori_loop(0, G // 16, vec_body, 0)

    def pass_body(p, _):
        sbase = p * NPR1
        gbase = wid * NPT + sbase
        pltpu.sync_copy(ad8_h.at[pl.ds(gbase, NPR1), :], adl)

        def zero_m(i, _):
            accm[pl.ds(i * 16, 16)] = zf
            return 0
        lax.fori_loop(0, NPR1 * 128 // 16, zero_m, 0)

        def zero_p(i, _):
            accp[pl.ds(i * 16, 16)] = zf
            return 0
        lax.fori_loop(0, NPR1, zero_p, 0)

        def chunk_body(c, carry):
            nselv, proc = carry
            pltpu.sync_copy(sel_h.at[pl.ds(wid * EPAD + c * CL, CL)], listb)

            def scan_body(v, nselv):
                pv = listb[pl.ds(v * 16, 16)]
                dl = pv & 2047
                sv = jnp.clip(lax.shift_right_logical(pv, 11), 0, N - 1)
                gpos = c * CL + v * 16 + iota
                within = ((dl >= sbase) & (dl < sbase + NPR1)
                          & (gpos < cntvec))
                incl = plsc.cumsum(within.astype(jnp.int32))
                pos = (nselv + incl - 1) & (CAPB - 1)
                plsc.store_scatter(sels, [pos], sv, mask=within)
                plsc.store_scatter(seld, [pos], dl - sbase, mask=within)
                return nselv + _lane_bcast(incl, 15)
            nselv = lax.fori_loop(0, CL // 16, scan_body, nselv)
            nsel = jnp.max(nselv)

            def drain(b, proc):
                process_batch(proc, nsel)
                return proc + G
            proc = lax.fori_loop(0, (nsel - proc) // G, drain, proc)
            return nselv, proc
        nselv, proc = lax.fori_loop(0, (cnt + CL - 1) // CL, chunk_body,
                                    (zi, 0))
        nsel = jnp.max(nselv)

        @pl.when(proc < nsel)
        def _():
            process_batch(proc, nsel)

        def fin_body(n, _):
            for j in range(H1):
                dn = plsc.load_gather(accp, [jnp.full((16,), 1, jnp.int32)
                                             * (n * 16 + j)])
                rv = 1.0 / (dn + 1e-16)
                bv = b1v[pl.ds(j * 16, 16)]
                val = accm[pl.ds(n * 128 + j * 16, 16)] * rv + bv
                res = jnp.where(val > 0, val, jnp.exp(val) - 1.0)
                accm[pl.ds(n * 128 + j * 16, 16)] = res
            return 0
        lax.fori_loop(0, NPR1, fin_body, 0)

        pltpu.sync_copy(accm, h2_h.at[pl.ds(gbase * 128, NPR1 * 128)])
        return 0
    lax.fori_loop(0, P1, pass_body, 0)


def _sc_layer2(sel_h, cnt_h, gcat_h, ad2_h, b2_h, out_h,
               acc, adl, b2v, listb, sidx, dlb, rows, cntv, sem):
    wid = lax.axis_index("s") * 2 + lax.axis_index("c")
    iota = lax.iota(jnp.int32, 16)
    zf = jnp.zeros((16,), jnp.float32)

    pltpu.sync_copy(b2_h, b2v)
    pltpu.sync_copy(cnt_h.at[pl.ds(wid * 16, 16)], cntv)
    cntvec = cntv[pl.ds(0, 16)]
    cnt = jnp.max(cntvec)
    pltpu.sync_copy(ad2_h.at[pl.ds(wid * NPT, NPT)], adl)

    def zero_a(i, _):
        acc[pl.ds(i * 16, 16)] = zf
        return 0
    lax.fori_loop(0, NPT, zero_a, 0)

    def chunk_body(c, _):
        pltpu.sync_copy(sel_h.at[pl.ds(wid * EPAD + c * CL, CL)], listb)

        def unpack_body(v, _):
            pv = listb[pl.ds(v * 16, 16)]
            sidx[pl.ds(v * 16, 16)] = jnp.clip(
                lax.shift_right_logical(pv, 11), 0, N - 1)
            dlb[pl.ds(v * 16, 16)] = pv & 2047
            return 0
        lax.fori_loop(0, CL // 16, unpack_body, 0)

        vn = jnp.clip(cnt - c * CL, 0, CL)

        def batch_body(b, _):
            gb = b * G
            pltpu.async_copy(gcat_h.at[sidx.at[pl.ds(gb, G)]], rows,
                             sem).wait()

            def vec_body(v, _):
                off = gb + v * 16
                dloc = jnp.clip(dlb[pl.ds(off, 16)], 0, NPT - 1)
                valid = (off + iota) < vn
                lid = iota + v * 16
                asv = plsc.load_gather(
                    rows, [lid, jnp.full((16,), 7, jnp.int32)])
                adv = plsc.load_gather(adl, [dloc])
                e = asv + adv
                e = jnp.where(e > 0, e, 0.2 * e)
                p = jnp.exp(e)
                plsc.addupdate_scatter(acc, [dloc * 16 + 7], p, mask=valid)
                for cc in range(C_OUT):
                    gv = plsc.load_gather(
                        rows, [lid, jnp.full((16,), cc, jnp.int32)])
                    plsc.addupdate_scatter(acc, [dloc * 16 + cc], p * gv,
                                           mask=valid)
                return 0
            lax.fori_loop(0, G // 16, vec_body, 0)
            return 0
        lax.fori_loop(0, (vn + G - 1) // G, batch_body, 0)
        return 0
    lax.fori_loop(0, (cnt + CL - 1) // CL, chunk_body, 0)

    bvec = b2v[pl.ds(0, 16)]

    def fin_body(n, _):
        dn = plsc.load_gather(acc, [jnp.full((16,), 1, jnp.int32)
                                    * (n * 16 + 7)])
        rv = 1.0 / (dn + 1e-16)
        acc[pl.ds(n * 16, 16)] = acc[pl.ds(n * 16, 16)] * rv + bvec
        return 0
    lax.fori_loop(0, NPT, fin_body, 0)

    pltpu.sync_copy(acc, out_h.at[pl.ds(wid * NPT * 16, NPT * 16)])


@functools.lru_cache(maxsize=None)
def _sc_calls():
    mesh = plsc.VectorSubcoreMesh(core_axis_name="c", subcore_axis_name="s")
    cp = pltpu.CompilerParams(needs_layout_passes=False)
    part = pl.kernel(
        _sc_partition,
        out_type=(jax.ShapeDtypeStruct((NWORK * EPAD,), jnp.int32),
                  jax.ShapeDtypeStruct((NWORK * 16,), jnp.int32)),
        mesh=mesh,
        scratch_types=[
            pltpu.VMEM((CHA,), jnp.int32),
            pltpu.VMEM((CHA,), jnp.int32),
            pltpu.VMEM((CAPA,), jnp.int32),
            pltpu.VMEM((16,), jnp.int32),
            pltpu.SemaphoreType.DMA,
        ],
        compiler_params=cp,
    )
    sc1 = pl.kernel(
        _sc_layer1,
        out_type=jax.ShapeDtypeStruct((NPAD * 128,), jnp.float32),
        mesh=mesh,
        scratch_types=[
            pltpu.VMEM((NPR1 * 128,), jnp.float32),
            pltpu.VMEM((NPR1 * 16,), jnp.float32),
            pltpu.VMEM((NPR1, 8), jnp.float32),
            pltpu.VMEM((128,), jnp.float32),
            pltpu.VMEM((CL,), jnp.int32),
            pltpu.VMEM((CAPB,), jnp.int32),
            pltpu.VMEM((CAPB,), jnp.int32),
            pltpu.VMEM((G, 256), jnp.float32),
            pltpu.VMEM((16,), jnp.int32),
            pltpu.SemaphoreType.DMA,
        ],
        compiler_params=cp,
    )
    sc2 = pl.kernel(
        _sc_layer2,
        out_type=jax.ShapeDtypeStruct((NPAD * 16,), jnp.float32),
        mesh=mesh,
        scratch_types=[
            pltpu.VMEM((NPT * 16,), jnp.float32),
            pltpu.VMEM((NPT,), jnp.float32),
            pltpu.VMEM((16,), jnp.float32),
            pltpu.VMEM((CL,), jnp.int32),
            pltpu.VMEM((CL,), jnp.int32),
            pltpu.VMEM((CL,), jnp.int32),
            pltpu.VMEM((G, 128), jnp.float32),
            pltpu.VMEM((16,), jnp.int32),
            pltpu.SemaphoreType.DMA,
        ],
        compiler_params=cp,
    )
    return part, sc1, sc2


def kernel(x, edge_index, W1, a_src1, a_dst1, b1, W2, a_src2, a_dst2, b2):
    src = edge_index[0]
    dst = edge_index[1]

    eye = jnp.eye(H1, dtype=jnp.float32)
    p_src = (eye[:, None, :] * a_src1[:, :, None]).reshape(H1 * C1, H1)
    p_dst = (eye[:, None, :] * a_dst1[:, :, None]).reshape(H1 * C1, H1)
    wcat1 = jnp.concatenate([W1, W1 @ p_src, W1 @ p_dst], axis=1)

    part_call, sc1_call, sc2_call = _sc_calls()

    selbuf, counts = part_call(src, dst)
    hcat = _tc_matmul(x, wcat1, 512, dpad=112)
    ad8 = jnp.pad(hcat[:, 136:144], ((0, NPAD - N), (0, 0)))
    h2_flat = sc1_call(selbuf, counts, hcat, ad8, b1)
    h2 = h2_flat.reshape(NPAD, 128)

    wcat2 = jnp.concatenate(
        [W2, W2 @ a_src2[0][:, None], W2 @ a_dst2[0][:, None],
         jnp.zeros((H1 * C1, 119), jnp.float32)], axis=1)
    gcat = _tc_matmul(h2, wcat2, 512)
    ad2 = gcat[:, 8]
    b2p = jnp.zeros((16,), jnp.float32).at[:C_OUT].set(b2)

    out_flat = sc2_call(selbuf, counts, gcat, ad2, b2p)
    return out_flat.reshape(NPAD, 16)[:N, :C_OUT]


# speedup vs baseline: 37.1398x; 1.8509x over previous
import functools

import jax
import jax.numpy as jnp
from jax import lax
from jax.experimental import pallas as pl
from jax.experimental.pallas import tpu as pltpu
from jax.experimental.pallas import tpu_sc as plsc

N = 50000
E = 800000
F_IN = 1433
H1 = 8
C1 = 16
C_OUT = 7

NWORK = 32
NPT = 1600
P1 = 5
NPR1 = 320
NPAD = 51200
CHA = 16000
CAPA = 32768
FB = 4096
EPAD = 806912
CL = 6400
CAPB = 8192
G = 64


def _lane_bcast(x, i):
    idx = jnp.full((16, 1), i, jnp.int32)
    dnums = lax.GatherDimensionNumbers(offset_dims=(), collapsed_slice_dims=(0,),
                                       start_index_map=(0,))
    return lax.gather(x, idx, dnums, (1,),
                      mode=lax.GatherScatterMode.PROMISE_IN_BOUNDS)


def _mm_kernel(x_ref, w_ref, o_ref, *, dpad):
    r = jnp.dot(x_ref[...], w_ref[...], preferred_element_type=jnp.float32)
    if dpad:
        r = jnp.concatenate(
            [r, jnp.zeros((r.shape[0], dpad), jnp.float32)], axis=1)
    o_ref[...] = r


def _tc_matmul(x, w, bm, dpad=0):
    m, k = x.shape
    _, n = w.shape
    grid = (m + bm - 1) // bm
    return pl.pallas_call(
        functools.partial(_mm_kernel, dpad=dpad),
        grid=(grid,),
        in_specs=[pl.BlockSpec((bm, k), lambda i: (i, 0)),
                  pl.BlockSpec((k, n), lambda i: (0, 0))],
        out_specs=pl.BlockSpec((bm, n + dpad), lambda i: (i, 0)),
        out_shape=jax.ShapeDtypeStruct((m, n + dpad), jnp.float32),
    )(x, w)


def _sc_partition(src_h, dst_h, sel_h, cnt_h, dbuf, sbuf, ring, cntv, sem):
    wid = lax.axis_index("s") * 2 + lax.axis_index("c")
    base = wid * NPT
    zi = jnp.zeros((16,), jnp.int32)

    def chunk_body(c, carry):
        nselv, flushed = carry
        pltpu.sync_copy(src_h.at[pl.ds(c * CHA, CHA)], sbuf)
        pltpu.sync_copy(dst_h.at[pl.ds(c * CHA, CHA)], dbuf)

        def scan_body(v, nselv):
            dv = dbuf[pl.ds(v * 16, 16)]
            within = (dv >= base) & (dv < base + NPT)
            sv = sbuf[pl.ds(v * 16, 16)]
            incl = plsc.cumsum(within.astype(jnp.int32))
            pos = (nselv + incl - 1) & (CAPA - 1)
            plsc.store_scatter(ring, [pos], sv * 2048 + (dv - base),
                               mask=within)
            return nselv + _lane_bcast(incl, 15)
        nselv = lax.fori_loop(0, CHA // 16, scan_body, nselv)
        nsel = jnp.max(nselv)

        def flush_body(f, flushed):
            fo = pl.multiple_of(flushed, FB)
            pltpu.sync_copy(ring.at[pl.ds(pl.multiple_of(fo & (CAPA - 1), FB), FB)],
                            sel_h.at[pl.ds(wid * EPAD + fo, FB)])
            return flushed + FB
        flushed = lax.fori_loop(0, (nsel - flushed) // FB, flush_body, flushed)
        return nselv, flushed
    nselv, flushed = lax.fori_loop(0, E // CHA, chunk_body, (zi, 0))
    nsel = jnp.max(nselv)

    def flush_tail(f, flushed):
        fo = pl.multiple_of(flushed, FB)
        pltpu.sync_copy(ring.at[pl.ds(pl.multiple_of(fo & (CAPA - 1), FB), FB)],
                        sel_h.at[pl.ds(wid * EPAD + fo, FB)])
        return flushed + FB
    lax.fori_loop(0, (nsel - flushed + FB - 1) // FB, flush_tail, flushed)

    cntv[pl.ds(0, 16)] = nselv
    pltpu.sync_copy(cntv, cnt_h.at[pl.ds(wid * 16, 16)])


def _sc_layer1(sel_h, cnt_h, hcat_h, ad8_h, b1_h, h2_h,
               accm, accp, adl, b1v, listb, sels, seld, rows, cntv, sem):
    wid = lax.axis_index("s") * 2 + lax.axis_index("c")
    iota = lax.iota(jnp.int32, 16)
    zf = jnp.zeros((16,), jnp.float32)
    zi = jnp.zeros((16,), jnp.int32)

    pltpu.sync_copy(b1_h, b1v)
    pltpu.sync_copy(cnt_h.at[pl.ds(wid * 16, 16)], cntv)
    cntvec = cntv[pl.ds(0, 16)]
    cnt = jnp.max(cntvec)

    def zero_sel(i, _):
        sels[pl.ds(i * 16, 16)] = zi
        seld[pl.ds(i * 16, 16)] = zi
        return 0
    lax.fori_loop(0, CAPB // 16, zero_sel, 0)

    def process_batch(proc, nsel):
        gb = pl.multiple_of(pl.multiple_of(proc, G) & (CAPB - 1), G)
        pltpu.async_copy(hcat_h.at[sels.at[pl.ds(gb, G)]], rows, sem).wait()
        lo8 = iota < 8

        def vec_body(v, _):
            off = gb + v * 16
            dloc = jnp.clip(seld[pl.ds(off, 16)], 0, NPR1 - 1)
            validi = ((proc + v * 16 + iota) < nsel).astype(jnp.int32)
            vbase = v * 16
            for ee in range(16):
                dv = _lane_bcast(dloc, ee)
                ve = _lane_bcast(validi, ee) > 0
                erow = jnp.full((16,), vbase + ee, jnp.int32)
                asv = plsc.load_gather(rows, [erow, 128 + iota])
                adv = plsc.load_gather(adl, [dv, iota & 7])
                eo = asv + adv
                eo = jnp.where(eo > 0, eo, 0.2 * eo)
                pe = jnp.exp(eo)
                plsc.addupdate_scatter(accp, [dv * 16 + iota], pe,
                                       mask=lo8 & ve)
                for j in range(H1):
                    pj = _lane_bcast(pe, j)
                    hv = plsc.load_gather(rows, [erow, j * 16 + iota])
                    plsc.addupdate_scatter(accm, [dv * 128 + j * 16 + iota],
                                           pj * hv, mask=ve)
            return 0
        lax.fori_loop(0, G // 16, vec_body, 0)

    def pass_body(p, _):
        sbase = p * NPR1
        gbase = wid * NPT + sbase
        pltpu.sync_copy(ad8_h.at[pl.ds(gbase, NPR1), :], adl)

        def zero_m(i, _):
            accm[pl.ds(i * 16, 16)] = zf
            return 0
        lax.fori_loop(0, NPR1 * 128 // 16, zero_m, 0)

        def zero_p(i, _):
            accp[pl.ds(i * 16, 16)] = zf
            return 0
        lax.fori_loop(0, NPR1, zero_p, 0)

        def chunk_body(c, carry):
            nselv, proc = carry
            pltpu.sync_copy(sel_h.at[pl.ds(wid * EPAD + c * CL, CL)], listb)

            def scan_body(v, nselv):
                pv = listb[pl.ds(v * 16, 16)]
                dl = pv & 2047
                sv = jnp.clip(lax.shift_right_logical(pv, 11), 0, N - 1)
                gpos = c * CL + v * 16 + iota
                within = ((dl >= sbase) & (dl < sbase + NPR1)
                          & (gpos < cntvec))
                incl = plsc.cumsum(within.astype(jnp.int32))
                pos = (nselv + incl - 1) & (CAPB - 1)
                plsc.store_scatter(sels, [pos], sv, mask=within)
                plsc.store_scatter(seld, [pos], dl - sbase, mask=within)
                return nselv + _lane_bcast(incl, 15)
            nselv = lax.fori_loop(0, CL // 16, scan_body, nselv)
            nsel = jnp.max(nselv)

            def drain(b, proc):
                process_batch(proc, nsel)
                return proc + G
            proc = lax.fori_loop(0, (nsel - proc) // G, drain, proc)
            return nselv, proc
        nselv, proc = lax.fori_loop(0, (cnt + CL - 1) // CL, chunk_body,
                                    (zi, 0))
        nsel = jnp.max(nselv)

        @pl.when(proc < nsel)
        def _():
            process_batch(proc, nsel)

        def fin_body(n, _):
            for j in range(H1):
                dn = plsc.load_gather(accp, [jnp.full((16,), 1, jnp.int32)
                                             * (n * 16 + j)])
                rv = 1.0 / (dn + 1e-16)
                bv = b1v[pl.ds(j * 16, 16)]
                val = accm[pl.ds(n * 128 + j * 16, 16)] * rv + bv
                res = jnp.where(val > 0, val, jnp.exp(val) - 1.0)
                accm[pl.ds(n * 128 + j * 16, 16)] = res
            return 0
        lax.fori_loop(0, NPR1, fin_body, 0)

        pltpu.sync_copy(accm, h2_h.at[pl.ds(gbase * 128, NPR1 * 128)])
        return 0
    lax.fori_loop(0, P1, pass_body, 0)


def _sc_layer2(sel_h, cnt_h, gcat_h, ad2_h, b2_h, out_h,
               acc, adl, b2v, listb, sidx, dlb, rows, cntv, sem):
    wid = lax.axis_index("s") * 2 + lax.axis_index("c")
    iota = lax.iota(jnp.int32, 16)
    zf = jnp.zeros((16,), jnp.float32)

    pltpu.sync_copy(b2_h, b2v)
    pltpu.sync_copy(cnt_h.at[pl.ds(wid * 16, 16)], cntv)
    cntvec = cntv[pl.ds(0, 16)]
    cnt = jnp.max(cntvec)
    pltpu.sync_copy(ad2_h.at[pl.ds(wid * NPT, NPT)], adl)

    def zero_a(i, _):
        acc[pl.ds(i * 16, 16)] = zf
        return 0
    lax.fori_loop(0, NPT, zero_a, 0)

    def chunk_body(c, _):
        pltpu.sync_copy(sel_h.at[pl.ds(wid * EPAD + c * CL, CL)], listb)

        def unpack_body(v, _):
            pv = listb[pl.ds(v * 16, 16)]
            sidx[pl.ds(v * 16, 16)] = jnp.clip(
                lax.shift_right_logical(pv, 11), 0, N - 1)
            dlb[pl.ds(v * 16, 16)] = pv & 2047
            return 0
        lax.fori_loop(0, CL // 16, unpack_body, 0)

        vn = jnp.clip(cnt - c * CL, 0, CL)

        def batch_body(b, _):
            gb = b * G
            pltpu.async_copy(gcat_h.at[sidx.at[pl.ds(gb, G)]], rows,
                             sem).wait()

            def vec_body(v, _):
                off = gb + v * 16
                dloc = jnp.clip(dlb[pl.ds(off, 16)], 0, NPT - 1)
                valid = (off + iota) < vn
                lid = iota + v * 16
                asv = plsc.load_gather(
                    rows, [lid, jnp.full((16,), 7, jnp.int32)])
                adv = plsc.load_gather(adl, [dloc])
                e = asv + adv
                e = jnp.where(e > 0, e, 0.2 * e)
                p = jnp.exp(e)
                plsc.addupdate_scatter(acc, [dloc * 16 + 7], p, mask=valid)
                for cc in range(C_OUT):
                    gv = plsc.load_gather(
                        rows, [lid, jnp.full((16,), cc, jnp.int32)])
                    plsc.addupdate_scatter(acc, [dloc * 16 + cc], p * gv,
                                           mask=valid)
                return 0
            lax.fori_loop(0, G // 16, vec_body, 0)
            return 0
        lax.fori_loop(0, (vn + G - 1) // G, batch_body, 0)
        return 0
    lax.fori_loop(0, (cnt + CL - 1) // CL, chunk_body, 0)

    bvec = b2v[pl.ds(0, 16)]

    def fin_body(n, _):
        dn = plsc.load_gather(acc, [jnp.full((16,), 1, jnp.int32)
                                    * (n * 16 + 7)])
        rv = 1.0 / (dn + 1e-16)
        acc[pl.ds(n * 16, 16)] = acc[pl.ds(n * 16, 16)] * rv + bvec
        return 0
    lax.fori_loop(0, NPT, fin_body, 0)

    pltpu.sync_copy(acc, out_h.at[pl.ds(wid * NPT * 16, NPT * 16)])


@functools.lru_cache(maxsize=None)
def _sc_calls():
    mesh = plsc.VectorSubcoreMesh(core_axis_name="c", subcore_axis_name="s")
    cp = pltpu.CompilerParams(needs_layout_passes=False)
    part = pl.kernel(
        _sc_partition,
        out_type=(jax.ShapeDtypeStruct((NWORK * EPAD,), jnp.int32),
                  jax.ShapeDtypeStruct((NWORK * 16,), jnp.int32)),
        mesh=mesh,
        scratch_types=[
            pltpu.VMEM((CHA,), jnp.int32),
            pltpu.VMEM((CHA,), jnp.int32),
            pltpu.VMEM((CAPA,), jnp.int32),
            pltpu.VMEM((16,), jnp.int32),
            pltpu.SemaphoreType.DMA,
        ],
        compiler_params=cp,
    )
    sc1 = pl.kernel(
        _sc_layer1,
        out_type=jax.ShapeDtypeStruct((NPAD * 128,), jnp.float32),
        mesh=mesh,
        scratch_types=[
            pltpu.VMEM((NPR1 * 128,), jnp.float32),
            pltpu.VMEM((NPR1 * 16,), jnp.float32),
            pltpu.VMEM((NPR1, 8), jnp.float32),
            pltpu.VMEM((128,), jnp.float32),
            pltpu.VMEM((CL,), jnp.int32),
            pltpu.VMEM((CAPB,), jnp.int32),
            pltpu.VMEM((CAPB,), jnp.int32),
            pltpu.VMEM((G, 256), jnp.float32),
            pltpu.VMEM((16,), jnp.int32),
            pltpu.SemaphoreType.DMA,
        ],
        compiler_params=cp,
    )
    sc2 = pl.kernel(
        _sc_layer2,
        out_type=jax.ShapeDtypeStruct((NPAD * 16,), jnp.float32),
        mesh=mesh,
        scratch_types=[
            pltpu.VMEM((NPT * 16,), jnp.float32),
            pltpu.VMEM((NPT,), jnp.float32),
            pltpu.VMEM((16,), jnp.float32),
            pltpu.VMEM((CL,), jnp.int32),
            pltpu.VMEM((CL,), jnp.int32),
            pltpu.VMEM((CL,), jnp.int32),
            pltpu.VMEM((G, 128), jnp.float32),
            pltpu.VMEM((16,), jnp.int32),
            pltpu.SemaphoreType.DMA,
        ],
        compiler_params=cp,
    )
    return part, sc1, sc2


def kernel(x, edge_index, W1, a_src1, a_dst1, b1, W2, a_src2, a_dst2, b2):
    src = edge_index[0]
    dst = edge_index[1]

    eye = jnp.eye(H1, dtype=jnp.float32)
    p_src = (eye[:, None, :] * a_src1[:, :, None]).reshape(H1 * C1, H1)
    p_dst = (eye[:, None, :] * a_dst1[:, :, None]).reshape(H1 * C1, H1)
    wcat1 = jnp.concatenate([W1, W1 @ p_src, W1 @ p_dst], axis=1)

    part_call, sc1_call, sc2_call = _sc_calls()

    selbuf, counts = part_call(src, dst)
    hcat = _tc_matmul(x, wcat1, 512, dpad=112)
    ad8 = jnp.pad(hcat[:, 136:144], ((0, NPAD - N), (0, 0)))
    h2_flat = sc1_call(selbuf, counts, hcat, ad8, b1)
    h2 = h2_flat.reshape(NPAD, 128)

    wcat2 = jnp.concatenate(
        [W2, W2 @ a_src2[0][:, None], W2 @ a_dst2[0][:, None],
         jnp.zeros((H1 * C1, 119), jnp.float32)], axis=1)
    gcat = _tc_matmul(h2, wcat2, 512)
    ad2 = gcat[:, 8]
    b2p = jnp.zeros((16,), jnp.float32).at[:C_OUT].set(b2)

    out_flat = sc2_call(selbuf, counts, gcat, ad2, b2p)
    return out_flat.reshape(NPAD, 16)[:N, :C_OUT]


# speedup vs baseline: 38.3096x; 1.0315x over previous
import functools

import jax
import jax.numpy as jnp
from jax import lax
from jax.experimental import pallas as pl
from jax.experimental.pallas import tpu as pltpu
from jax.experimental.pallas import tpu_sc as plsc

N = 50000
E = 800000
F_IN = 1433
H1 = 8
C1 = 16
C_OUT = 7

NWORK = 32
NPT = 1600
P1 = 8
NPR1 = 200
NPAD = 51200
CHA = 16000
CAPA = 32768
FB = 4096
EPAD = 806912
CL = 6400
CAPB = 8192
G = 128


def _lane_bcast(x, i):
    idx = jnp.full((16, 1), i, jnp.int32)
    dnums = lax.GatherDimensionNumbers(offset_dims=(), collapsed_slice_dims=(0,),
                                       start_index_map=(0,))
    return lax.gather(x, idx, dnums, (1,),
                      mode=lax.GatherScatterMode.PROMISE_IN_BOUNDS)


def _mm_kernel(x_ref, w_ref, o_ref, *, dpad):
    r = jnp.dot(x_ref[...], w_ref[...], preferred_element_type=jnp.float32)
    if dpad:
        r = jnp.concatenate(
            [r, jnp.zeros((r.shape[0], dpad), jnp.float32)], axis=1)
    o_ref[...] = r


def _tc_matmul(x, w, bm, dpad=0):
    m, k = x.shape
    _, n = w.shape
    grid = (m + bm - 1) // bm
    return pl.pallas_call(
        functools.partial(_mm_kernel, dpad=dpad),
        grid=(grid,),
        in_specs=[pl.BlockSpec((bm, k), lambda i: (i, 0)),
                  pl.BlockSpec((k, n), lambda i: (0, 0))],
        out_specs=pl.BlockSpec((bm, n + dpad), lambda i: (i, 0)),
        out_shape=jax.ShapeDtypeStruct((m, n + dpad), jnp.float32),
    )(x, w)


def _sc_partition(src_h, dst_h, sel_h, cnt_h, dbuf, sbuf, ring, cntv, sem):
    wid = lax.axis_index("s") * 2 + lax.axis_index("c")
    base = wid * NPT
    zi = jnp.zeros((16,), jnp.int32)

    def chunk_body(c, carry):
        nselv, flushed = carry
        pltpu.sync_copy(src_h.at[pl.ds(c * CHA, CHA)], sbuf)
        pltpu.sync_copy(dst_h.at[pl.ds(c * CHA, CHA)], dbuf)

        def scan_body(v, nselv):
            dv = dbuf[pl.ds(v * 16, 16)]
            within = (dv >= base) & (dv < base + NPT)
            sv = sbuf[pl.ds(v * 16, 16)]
            incl = plsc.cumsum(within.astype(jnp.int32))
            pos = (nselv + incl - 1) & (CAPA - 1)
            plsc.store_scatter(ring, [pos], sv * 2048 + (dv - base),
                               mask=within)
            return nselv + _lane_bcast(incl, 15)
        nselv = lax.fori_loop(0, CHA // 16, scan_body, nselv)
        nsel = jnp.max(nselv)

        def flush_body(f, flushed):
            fo = pl.multiple_of(flushed, FB)
            pltpu.sync_copy(ring.at[pl.ds(pl.multiple_of(fo & (CAPA - 1), FB), FB)],
                            sel_h.at[pl.ds(wid * EPAD + fo, FB)])
            return flushed + FB
        flushed = lax.fori_loop(0, (nsel - flushed) // FB, flush_body, flushed)
        return nselv, flushed
    nselv, flushed = lax.fori_loop(0, E // CHA, chunk_body, (zi, 0))
    nsel = jnp.max(nselv)

    def flush_tail(f, flushed):
        fo = pl.multiple_of(flushed, FB)
        pltpu.sync_copy(ring.at[pl.ds(pl.multiple_of(fo & (CAPA - 1), FB), FB)],
                        sel_h.at[pl.ds(wid * EPAD + fo, FB)])
        return flushed + FB
    lax.fori_loop(0, (nsel - flushed + FB - 1) // FB, flush_tail, flushed)

    cntv[pl.ds(0, 16)] = nselv
    pltpu.sync_copy(cntv, cnt_h.at[pl.ds(wid * 16, 16)])


def _sc_layer1(sel_h, cnt_h, hcat_h, ad8_h, b1_h, h2_h,
               accm, accp, adl, b1v, listb, sels, seld, rows, cntv, sem):
    wid = lax.axis_index("s") * 2 + lax.axis_index("c")
    iota = lax.iota(jnp.int32, 16)
    zf = jnp.zeros((16,), jnp.float32)
    zi = jnp.zeros((16,), jnp.int32)

    pltpu.sync_copy(b1_h, b1v)
    pltpu.sync_copy(cnt_h.at[pl.ds(wid * 16, 16)], cntv)
    cntvec = cntv[pl.ds(0, 16)]
    cnt = jnp.max(cntvec)

    def zero_sel(i, _):
        sels[pl.ds(i * 16, 16)] = zi
        seld[pl.ds(i * 16, 16)] = zi
        return 0
    lax.fori_loop(0, CAPB // 16, zero_sel, 0)

    def process_batch(proc, nsel):
        gb = pl.multiple_of(pl.multiple_of(proc, G) & (CAPB - 1), G)
        pltpu.async_copy(hcat_h.at[sels.at[pl.ds(gb, G)]], rows, sem).wait()
        lo8 = iota < 8

        def vec_body(v, _):
            off = gb + v * 16
            dloc = jnp.clip(seld[pl.ds(off, 16)], 0, NPR1 - 1)
            validi = ((proc + v * 16 + iota) < nsel).astype(jnp.int32)
            vbase = v * 16
            for ee in range(16):
                dv = _lane_bcast(dloc, ee)
                ve = _lane_bcast(validi, ee) > 0
                erow = jnp.full((16,), vbase + ee, jnp.int32)
                asv = plsc.load_gather(rows, [erow, 128 + iota])
                adv = plsc.load_gather(adl, [dv, iota & 7])
                eo = asv + adv
                eo = jnp.where(eo > 0, eo, 0.2 * eo)
                pe = jnp.exp(eo)
                plsc.addupdate_scatter(accp, [dv * 16 + iota], pe,
                                       mask=lo8 & ve)
                for j in range(H1):
                    pj = _lane_bcast(pe, j)
                    hv = plsc.load_gather(rows, [erow, j * 16 + iota])
                    plsc.addupdate_scatter(accm, [dv * 128 + j * 16 + iota],
                                           pj * hv, mask=ve)
            return 0
        lax.fori_loop(0, G // 16, vec_body, 0)

    def pass_body(p, _):
        sbase = p * NPR1
        gbase = wid * NPT + sbase
        pltpu.sync_copy(ad8_h.at[pl.ds(gbase, NPR1), :], adl)

        def zero_m(i, _):
            accm[pl.ds(i * 16, 16)] = zf
            return 0
        lax.fori_loop(0, NPR1 * 128 // 16, zero_m, 0)

        def zero_p(i, _):
            accp[pl.ds(i * 16, 16)] = zf
            return 0
        lax.fori_loop(0, NPR1, zero_p, 0)

        def chunk_body(c, carry):
            nselv, proc = carry
            pltpu.sync_copy(sel_h.at[pl.ds(wid * EPAD + c * CL, CL)], listb)

            def scan_body(v, nselv):
                pv = listb[pl.ds(v * 16, 16)]
                dl = pv & 2047
                sv = jnp.clip(lax.shift_right_logical(pv, 11), 0, N - 1)
                gpos = c * CL + v * 16 + iota
                within = ((dl >= sbase) & (dl < sbase + NPR1)
                          & (gpos < cntvec))
                incl = plsc.cumsum(within.astype(jnp.int32))
                pos = (nselv + incl - 1) & (CAPB - 1)
                plsc.store_scatter(sels, [pos], sv, mask=within)
                plsc.store_scatter(seld, [pos], dl - sbase, mask=within)
                return nselv + _lane_bcast(incl, 15)
            nselv = lax.fori_loop(0, CL // 16, scan_body, nselv)
            nsel = jnp.max(nselv)

            def drain(b, proc):
                process_batch(proc, nsel)
                return proc + G
            proc = lax.fori_loop(0, (nsel - proc) // G, drain, proc)
            return nselv, proc
        nselv, proc = lax.fori_loop(0, (cnt + CL - 1) // CL, chunk_body,
                                    (zi, 0))
        nsel = jnp.max(nselv)

        @pl.when(proc < nsel)
        def _():
            process_batch(proc, nsel)

        def fin_body(n, _):
            for j in range(H1):
                dn = plsc.load_gather(accp, [jnp.full((16,), 1, jnp.int32)
                                             * (n * 16 + j)])
                rv = 1.0 / (dn + 1e-16)
                bv = b1v[pl.ds(j * 16, 16)]
                val = accm[pl.ds(n * 128 + j * 16, 16)] * rv + bv
                res = jnp.where(val > 0, val, jnp.exp(val) - 1.0)
                accm[pl.ds(n * 128 + j * 16, 16)] = res
            return 0
        lax.fori_loop(0, NPR1, fin_body, 0)

        pltpu.sync_copy(accm, h2_h.at[pl.ds(gbase * 128, NPR1 * 128)])
        return 0
    lax.fori_loop(0, P1, pass_body, 0)


def _sc_layer2(sel_h, cnt_h, gcat_h, ad2_h, b2_h, out_h,
               acc, adl, b2v, listb, sidx, dlb, rows, cntv, sem):
    wid = lax.axis_index("s") * 2 + lax.axis_index("c")
    iota = lax.iota(jnp.int32, 16)
    zf = jnp.zeros((16,), jnp.float32)

    pltpu.sync_copy(b2_h, b2v)
    pltpu.sync_copy(cnt_h.at[pl.ds(wid * 16, 16)], cntv)
    cntvec = cntv[pl.ds(0, 16)]
    cnt = jnp.max(cntvec)
    pltpu.sync_copy(ad2_h.at[pl.ds(wid * NPT, NPT)], adl)

    def zero_a(i, _):
        acc[pl.ds(i * 16, 16)] = zf
        return 0
    lax.fori_loop(0, NPT, zero_a, 0)

    def chunk_body(c, _):
        pltpu.sync_copy(sel_h.at[pl.ds(wid * EPAD + c * CL, CL)], listb)

        def unpack_body(v, _):
            pv = listb[pl.ds(v * 16, 16)]
            sidx[pl.ds(v * 16, 16)] = jnp.clip(
                lax.shift_right_logical(pv, 11), 0, N - 1)
            dlb[pl.ds(v * 16, 16)] = pv & 2047
            return 0
        lax.fori_loop(0, CL // 16, unpack_body, 0)

        vn = jnp.clip(cnt - c * CL, 0, CL)

        def batch_body(b, _):
            gb = b * G
            pltpu.async_copy(gcat_h.at[sidx.at[pl.ds(gb, G)]], rows,
                             sem).wait()

            lo8 = iota < 8

            def vec_body(v, _):
                off = gb + v * 16
                dloc = jnp.clip(dlb[pl.ds(off, 16)], 0, NPT - 1)
                validi = ((off + iota) < vn).astype(jnp.int32)
                lid = iota + v * 16
                asv = plsc.load_gather(
                    rows, [lid, jnp.full((16,), 7, jnp.int32)])
                adv = plsc.load_gather(adl, [dloc])
                e = asv + adv
                e = jnp.where(e > 0, e, 0.2 * e)
                p = jnp.exp(e)
                for ee in range(16):
                    dv = _lane_bcast(dloc, ee)
                    ve = _lane_bcast(validi, ee) > 0
                    erow = jnp.full((16,), v * 16 + ee, jnp.int32)
                    gv = plsc.load_gather(rows, [erow, iota])
                    pj = _lane_bcast(p, ee)
                    val = jnp.where(iota == 7, pj, pj * gv)
                    plsc.addupdate_scatter(acc, [dv * 16 + iota], val,
                                           mask=lo8 & ve)
                return 0
            lax.fori_loop(0, G // 16, vec_body, 0)
            return 0
        lax.fori_loop(0, (vn + G - 1) // G, batch_body, 0)
        return 0
    lax.fori_loop(0, (cnt + CL - 1) // CL, chunk_body, 0)

    bvec = b2v[pl.ds(0, 16)]

    def fin_body(n, _):
        dn = plsc.load_gather(acc, [jnp.full((16,), 1, jnp.int32)
                                    * (n * 16 + 7)])
        rv = 1.0 / (dn + 1e-16)
        acc[pl.ds(n * 16, 16)] = acc[pl.ds(n * 16, 16)] * rv + bvec
        return 0
    lax.fori_loop(0, NPT, fin_body, 0)

    pltpu.sync_copy(acc, out_h.at[pl.ds(wid * NPT * 16, NPT * 16)])


@functools.lru_cache(maxsize=None)
def _sc_calls():
    mesh = plsc.VectorSubcoreMesh(core_axis_name="c", subcore_axis_name="s")
    cp = pltpu.CompilerParams(needs_layout_passes=False)
    part = pl.kernel(
        _sc_partition,
        out_type=(jax.ShapeDtypeStruct((NWORK * EPAD,), jnp.int32),
                  jax.ShapeDtypeStruct((NWORK * 16,), jnp.int32)),
        mesh=mesh,
        scratch_types=[
            pltpu.VMEM((CHA,), jnp.int32),
            pltpu.VMEM((CHA,), jnp.int32),
            pltpu.VMEM((CAPA,), jnp.int32),
            pltpu.VMEM((16,), jnp.int32),
            pltpu.SemaphoreType.DMA,
        ],
        compiler_params=cp,
    )
    sc1 = pl.kernel(
        _sc_layer1,
        out_type=jax.ShapeDtypeStruct((NPAD * 128,), jnp.float32),
        mesh=mesh,
        scratch_types=[
            pltpu.VMEM((NPR1 * 128,), jnp.float32),
            pltpu.VMEM((NPR1 * 16,), jnp.float32),
            pltpu.VMEM((NPR1, 8), jnp.float32),
            pltpu.VMEM((128,), jnp.float32),
            pltpu.VMEM((CL,), jnp.int32),
            pltpu.VMEM((CAPB,), jnp.int32),
            pltpu.VMEM((CAPB,), jnp.int32),
            pltpu.VMEM((G, 256), jnp.float32),
            pltpu.VMEM((16,), jnp.int32),
            pltpu.SemaphoreType.DMA,
        ],
        compiler_params=cp,
    )
    sc2 = pl.kernel(
        _sc_layer2,
        out_type=jax.ShapeDtypeStruct((NPAD * 16,), jnp.float32),
        mesh=mesh,
        scratch_types=[
            pltpu.VMEM((NPT * 16,), jnp.float32),
            pltpu.VMEM((NPT,), jnp.float32),
            pltpu.VMEM((16,), jnp.float32),
            pltpu.VMEM((CL,), jnp.int32),
            pltpu.VMEM((CL,), jnp.int32),
            pltpu.VMEM((CL,), jnp.int32),
            pltpu.VMEM((G, 128), jnp.float32),
            pltpu.VMEM((16,), jnp.int32),
            pltpu.SemaphoreType.DMA,
        ],
        compiler_params=cp,
    )
    return part, sc1, sc2


def kernel(x, edge_index, W1, a_src1, a_dst1, b1, W2, a_src2, a_dst2, b2):
    src = edge_index[0]
    dst = edge_index[1]

    eye = jnp.eye(H1, dtype=jnp.float32)
    p_src = (eye[:, None, :] * a_src1[:, :, None]).reshape(H1 * C1, H1)
    p_dst = (eye[:, None, :] * a_dst1[:, :, None]).reshape(H1 * C1, H1)
    wcat1 = jnp.concatenate([W1, W1 @ p_src, W1 @ p_dst], axis=1)

    part_call, sc1_call, sc2_call = _sc_calls()

    selbuf, counts = part_call(src, dst)
    hcat = _tc_matmul(x, wcat1, 512, dpad=112)
    ad8 = jnp.pad(hcat[:, 136:144], ((0, NPAD - N), (0, 0)))
    h2_flat = sc1_call(selbuf, counts, hcat, ad8, b1)
    h2 = h2_flat.reshape(NPAD, 128)

    wcat2 = jnp.concatenate(
        [W2, W2 @ a_src2[0][:, None], W2 @ a_dst2[0][:, None],
         jnp.zeros((H1 * C1, 119), jnp.float32)], axis=1)
    gcat = _tc_matmul(h2, wcat2, 512)
    ad2 = gcat[:, 8]
    b2p = jnp.zeros((16,), jnp.float32).at[:C_OUT].set(b2)

    out_flat = sc2_call(selbuf, counts, gcat, ad2, b2p)
    return out_flat.reshape(NPAD, 16)[:N, :C_OUT]
